# Initial kernel scaffold; baseline (speedup 1.0000x reference)
import jax
import jax.numpy as jnp
from jax.experimental import pallas as pl


def kernel(embedding, edge_index, edge_weight):
    raise NotImplementedError("write your pallas kernel here")


# speedup vs baseline: 2.8117x; 2.8117x over previous
import functools

import jax
import jax.numpy as jnp
from jax import lax
from jax.experimental import pallas as pl
from jax.experimental.pallas import tpu as pltpu, tpu_sc as plsc

N_NODE = 10000
NPAD = 10240
EMB = 112
DPAD = 128
DH = DPAD // 2
N_EDGE = 640000
NC = 2
NS = 16
L = 16
EPS = N_EDGE // NS
B = 80
NCH = EPS // B
RPT = NPAD // NS


def _body(xh, rows, cols, w, out, x1, xnew, colsv, rowsv, wv, G, bufA, bufB, sem):
    cid = lax.axis_index("c")
    sid = lax.axis_index("s")
    half_base = cid * NPAD
    stripe = half_base + sid * RPT
    sstripe = sid * RPT
    ebase = sid * EPS

    def zero_bufA():
        zz = jnp.zeros((L,), jnp.float32)
        def zb(i, c):
            for j in range(DH // L):
                bufA[i, pl.ds(j * L, L)] = zz
            return c
        lax.fori_loop(0, RPT, zb, 0)

    def zero_xnew():
        pltpu.sync_copy(bufA, xnew.at[pl.ds(sstripe, RPT)])

    def edge_pass(src_hbm):
        def chunk(ci, c):
            base = ebase + ci * B
            pltpu.sync_copy(cols.at[pl.ds(base, B)], colsv)
            pltpu.sync_copy(rows.at[pl.ds(base, B)], rowsv)
            pltpu.sync_copy(w.at[pl.ds(base, B)], wv)
            for j in range(B // L):
                s = pl.ds(j * L, L)
                colsv[s] = colsv[s] + half_base
            pltpu.async_copy(src_hbm.at[colsv], G, sem).wait()
            def scale(e, c2):
                idx = jnp.zeros((L,), jnp.int32) + e
                wvec = plsc.load_gather(wv, [idx])
                for j in range(DH // L):
                    s = pl.ds(j * L, L)
                    G[e, s] = G[e, s] * wvec
                return c2
            lax.fori_loop(0, B, scale, 0)
            pltpu.sync_copy(G, xnew.at[rowsv], add=True)
            return c
        lax.fori_loop(0, NCH, chunk, 0)

    zero_bufA()
    zero_xnew()
    plsc.subcore_barrier()
    edge_pass(xh)
    plsc.subcore_barrier()

    pltpu.sync_copy(xnew.at[pl.ds(sstripe, RPT)], bufB)
    pltpu.sync_copy(bufB, x1.at[pl.ds(stripe, RPT)])
    zero_xnew()
    plsc.subcore_barrier()

    edge_pass(x1)
    plsc.subcore_barrier()

    pltpu.sync_copy(xh.at[pl.ds(stripe, RPT)], bufA)
    def addb(i, c):
        for j in range(DH // L):
            s = pl.ds(j * L, L)
            bufA[i, s] = bufA[i, s] + bufB[i, s]
        return c
    lax.fori_loop(0, RPT, addb, 0)
    pltpu.sync_copy(xnew.at[pl.ds(sstripe, RPT)], bufB)
    def fin(i, c):
        for j in range(DH // L):
            s = pl.ds(j * L, L)
            bufA[i, s] = (bufA[i, s] + bufB[i, s]) * (1.0 / 3.0)
        return c
    lax.fori_loop(0, RPT, fin, 0)
    pltpu.sync_copy(bufA, out.at[pl.ds(stripe, RPT)])


@jax.jit
def kernel(embedding, edge_index, edge_weight):
    xpad = jnp.pad(embedding, ((0, NPAD - N_NODE), (0, DPAD - EMB)))
    xh = jnp.concatenate([xpad[:, :DH], xpad[:, DH:]], axis=0)
    rows = edge_index[0]
    cols = edge_index[1]

    f32 = jnp.float32
    run = pl.kernel(
        _body,
        out_type=(
            jax.ShapeDtypeStruct((NC * NPAD, DH), f32),
            jax.ShapeDtypeStruct((NC * NPAD, DH), f32),
        ),
        mesh=plsc.VectorSubcoreMesh(
            core_axis_name="c", subcore_axis_name="s",
            num_cores=NC, num_subcores=NS),
        compiler_params=pltpu.CompilerParams(
            use_tc_tiling_on_sc=False, needs_layout_passes=False),
        scratch_types=[
            pltpu.VMEM_SHARED((NPAD, DH), f32),
            pltpu.VMEM((B,), jnp.int32),
            pltpu.VMEM((B,), jnp.int32),
            pltpu.VMEM((B,), f32),
            pltpu.VMEM((B, DH), f32),
            pltpu.VMEM((RPT, DH), f32),
            pltpu.VMEM((RPT, DH), f32),
            pltpu.SemaphoreType.DMA,
        ],
    )
    o, _x1 = run(xh, rows, cols, edge_weight)
    return jnp.concatenate([o[:N_NODE], o[NPAD:NPAD + N_NODE]], axis=1)[:, :EMB]


# speedup vs baseline: 5.8777x; 2.0904x over previous
import functools

import jax
import jax.numpy as jnp
from jax import lax
from jax.experimental import pallas as pl
from jax.experimental.pallas import tpu as pltpu, tpu_sc as plsc

N_NODE = 10000
NPAD = 10240
EMB = 112
DPAD = 128
DH = DPAD // 2
N_EDGE = 640000
NC = 2
NS = 16
L = 16
EPS = N_EDGE // NS
B = 80
NCH = EPS // B
RPT = NPAD // NS
SBUF = RPT // 2


def _body(xh, rows, cols, w, out, x1, xnew,
          colsv0, rowsv0, wv0, G0, colsv1, rowsv1, wv1, G1,
          bufA, bufB, semi0, semi1, semg0, semg1):
    cid = lax.axis_index("c")
    sid = lax.axis_index("s")
    half_base = cid * NPAD
    stripe = half_base + sid * RPT
    sstripe = sid * RPT
    ebase = sid * EPS

    bufs = ((colsv0, rowsv0, wv0, G0, semi0, semg0),
            (colsv1, rowsv1, wv1, G1, semi1, semg1))

    def zero_bufA():
        zz = jnp.zeros((L,), jnp.float32)
        def zb(i, c):
            for j in range(DH // L):
                bufA[i, pl.ds(j * L, L)] = zz
            return c
        lax.fori_loop(0, SBUF, zb, 0)

    def zero_xnew():
        for p in range(RPT // SBUF):
            pltpu.sync_copy(bufA, xnew.at[pl.ds(sstripe + p * SBUF, SBUF)])

    def start_idx(ci, p):
        colsv, rowsv, wv, _, semi, _ = bufs[p]
        base = ebase + ci * B
        pltpu.async_copy(cols.at[pl.ds(base, B)], colsv, semi)
        pltpu.async_copy(rows.at[pl.ds(base, B)], rowsv, semi)
        pltpu.async_copy(w.at[pl.ds(base, B)], wv, semi)

    def wait_idx(p):
        colsv, rowsv, wv, _, semi, _ = bufs[p]
        pltpu.make_async_copy(cols.at[pl.ds(0, B)], colsv, semi).wait()
        pltpu.make_async_copy(rows.at[pl.ds(0, B)], rowsv, semi).wait()
        pltpu.make_async_copy(w.at[pl.ds(0, B)], wv, semi).wait()

    def start_gather(src_hbm, p):
        colsv, _, _, G, _, semg = bufs[p]
        for j in range(B // L):
            s = pl.ds(j * L, L)
            colsv[s] = colsv[s] + half_base
        pltpu.async_copy(src_hbm.at[colsv], G, semg)

    def wait_gather(src_hbm, p):
        colsv, _, _, G, _, semg = bufs[p]
        pltpu.make_async_copy(src_hbm.at[colsv], G, semg).wait()

    def process(src_hbm, p):
        _, rowsv, wv, G, _, _ = bufs[p]
        def scale(e, c2):
            idx = jnp.zeros((L,), jnp.int32) + e
            wvec = plsc.load_gather(wv, [idx])
            for j in range(DH // L):
                s = pl.ds(j * L, L)
                G[e, s] = G[e, s] * wvec
            return c2
        lax.fori_loop(0, B, scale, 0)
        pltpu.sync_copy(G, xnew.at[rowsv], add=True)

    def edge_pass(src_hbm):
        start_idx(0, 0)
        wait_idx(0)
        start_gather(src_hbm, 0)
        start_idx(1, 1)

        def step(k, c):
            nxt0 = lax.rem(2 * k + 2, NCH)
            nxt1 = lax.rem(2 * k + 3, NCH)
            wait_gather(src_hbm, 0)
            wait_idx(1)
            start_gather(src_hbm, 1)
            process(src_hbm, 0)
            start_idx(nxt0, 0)
            wait_gather(src_hbm, 1)
            wait_idx(0)
            start_gather(src_hbm, 0)
            process(src_hbm, 1)
            start_idx(nxt1, 1)
            return c
        lax.fori_loop(0, NCH // 2, step, 0)
        wait_gather(src_hbm, 0)
        wait_idx(1)

    zero_bufA()
    zero_xnew()
    plsc.subcore_barrier()
    edge_pass(xh)
    plsc.subcore_barrier()

    for p in range(RPT // SBUF):
        pltpu.sync_copy(xnew.at[pl.ds(sstripe + p * SBUF, SBUF)], bufB)
        pltpu.sync_copy(bufB, x1.at[pl.ds(stripe + p * SBUF, SBUF)])
    zero_xnew()
    plsc.subcore_barrier()

    edge_pass(x1)
    plsc.subcore_barrier()

    for p in range(RPT // SBUF):
        pltpu.sync_copy(xh.at[pl.ds(stripe + p * SBUF, SBUF)], bufA)
        pltpu.sync_copy(x1.at[pl.ds(stripe + p * SBUF, SBUF)], bufB)
        def addb(i, c):
            for j in range(DH // L):
                s = pl.ds(j * L, L)
                bufA[i, s] = bufA[i, s] + bufB[i, s]
            return c
        lax.fori_loop(0, SBUF, addb, 0)
        pltpu.sync_copy(xnew.at[pl.ds(sstripe + p * SBUF, SBUF)], bufB)
        def fin(i, c):
            for j in range(DH // L):
                s = pl.ds(j * L, L)
                bufA[i, s] = (bufA[i, s] + bufB[i, s]) * (1.0 / 3.0)
            return c
        lax.fori_loop(0, SBUF, fin, 0)
        pltpu.sync_copy(bufA, out.at[pl.ds(stripe + p * SBUF, SBUF)])


@jax.jit
def kernel(embedding, edge_index, edge_weight):
    xpad = jnp.pad(embedding, ((0, NPAD - N_NODE), (0, DPAD - EMB)))
    xh = jnp.concatenate([xpad[:, :DH], xpad[:, DH:]], axis=0)
    rows = edge_index[0]
    cols = edge_index[1]

    f32 = jnp.float32
    run = pl.kernel(
        _body,
        out_type=(
            jax.ShapeDtypeStruct((NC * NPAD, DH), f32),
            jax.ShapeDtypeStruct((NC * NPAD, DH), f32),
        ),
        mesh=plsc.VectorSubcoreMesh(
            core_axis_name="c", subcore_axis_name="s",
            num_cores=NC, num_subcores=NS),
        compiler_params=pltpu.CompilerParams(
            use_tc_tiling_on_sc=False, needs_layout_passes=False),
        scratch_types=[
            pltpu.VMEM_SHARED((NPAD, DH), f32),
            pltpu.VMEM((B,), jnp.int32),
            pltpu.VMEM((B,), jnp.int32),
            pltpu.VMEM((B,), f32),
            pltpu.VMEM((B, DH), f32),
            pltpu.VMEM((B,), jnp.int32),
            pltpu.VMEM((B,), jnp.int32),
            pltpu.VMEM((B,), f32),
            pltpu.VMEM((B, DH), f32),
            pltpu.VMEM((SBUF, DH), f32),
            pltpu.VMEM((SBUF, DH), f32),
            pltpu.SemaphoreType.DMA,
            pltpu.SemaphoreType.DMA,
            pltpu.SemaphoreType.DMA,
            pltpu.SemaphoreType.DMA,
        ],
    )
    o, _x1 = run(xh, rows, cols, edge_weight)
    return jnp.concatenate([o[:N_NODE], o[NPAD:NPAD + N_NODE]], axis=1)[:, :EMB]
